# baseline (speedup 1.0000x reference)
import functools

import jax
import jax.numpy as jnp
from jax import lax
from jax.experimental import pallas as pl
from jax.experimental.pallas import tpu as pltpu
from jax.experimental.pallas import tpu_sc as plsc

_K1 = 1.5
_B = 0.75
_N = 50000
_V = 1000
_Q = 32
_L = 16
_TOPK = 10


_CB = 128
_NCB = -(-_V // _CB)
_TBH = [_CB] * (_NCB - 1) + [_V - _CB * (_NCB - 1)]
_RCH = 8192
_NRCH = _N // _RCH
_TAILR = _N - _NRCH * _RCH


def _fused_body(tft_hbm, dl_ref, cnt_ref, avg_ref, out_ref, bufa, bufb, sems):
    bufs = [bufa, bufb]

    def bcopy(c, b):
        h = _TBH[c]
        src = tft_hbm.at[pl.ds(c * _CB, h), :]
        dst = bufs[b] if h == _CB else bufs[b].at[pl.ds(0, h), :]
        return pltpu.make_async_copy(src, dst, sems.at[b])

    bcopy(0, 0).start()
    avg = avg_ref[0, 0]

    for c in range(_NCB):
        h = _TBH[c]
        cur = bufs[c % 2]
        bcopy(c, c % 2).wait()
        if c + 1 < _NCB:
            bcopy(c + 1, (c + 1) % 2).start()

        def df_chunk(i, d):
            blk = cur[0:h, pl.ds(i * _RCH, _RCH)]
            return d + jnp.sum((blk > 0).astype(jnp.float32), axis=1,
                               keepdims=True)

        df = lax.fori_loop(0, _NRCH, df_chunk,
                           jnp.zeros((h, 1), jnp.float32))
        blk = cur[0:h, pl.ds(_NRCH * _RCH, _TAILR)]
        df += jnp.sum((blk > 0).astype(jnp.float32), axis=1, keepdims=True)

        idf = jnp.log((_N - df + 0.5) / (df + 0.5))
        cnt = cnt_ref[c][:, 0:h]
        cnt_bf = cnt.astype(jnp.bfloat16)

        def _dot(x):
            return lax.dot_general(
                cnt_bf, x, (((1,), (0,)), ((), ())),
                preferred_element_type=jnp.float32)

        def sc_chunk(r0, rn):
            tfb = cur[0:h, pl.ds(r0, rn)]
            dlr = dl_ref[0:1, pl.ds(r0, rn)]
            norm = _K1 * (1.0 - _B + _B * dlr / avg)
            num = tfb * (_K1 + 1.0)
            ap = idf * num / (tfb + norm)
            hi = ap.astype(jnp.bfloat16)
            r1 = ap - hi.astype(jnp.float32)
            mid = r1.astype(jnp.bfloat16)
            lo = (r1 - mid.astype(jnp.float32)).astype(jnp.bfloat16)
            return _dot(hi) + _dot(mid) + _dot(lo)

        if c == 0:
            def body0(i, _):
                r0 = i * _RCH
                out_ref[:, pl.ds(r0, _RCH)] = sc_chunk(r0, _RCH)
                return 0
            lax.fori_loop(0, _NRCH, body0, 0)
            out_ref[:, pl.ds(_NRCH * _RCH, _TAILR)] = sc_chunk(
                _NRCH * _RCH, _TAILR)
        else:
            def bodyn(i, _):
                r0 = i * _RCH
                out_ref[:, pl.ds(r0, _RCH)] += sc_chunk(r0, _RCH)
                return 0
            lax.fori_loop(0, _NRCH, bodyn, 0)
            out_ref[:, pl.ds(_NRCH * _RCH, _TAILR)] += sc_chunk(
                _NRCH * _RCH, _TAILR)


def _fused_pass(tft, dl_row, counts_blocks, avg):
    return pl.pallas_call(
        _fused_body,
        in_specs=[
            pl.BlockSpec(memory_space=pl.ANY),
            pl.BlockSpec(memory_space=pltpu.MemorySpace.VMEM),
            pl.BlockSpec(memory_space=pltpu.MemorySpace.VMEM),
            pl.BlockSpec(memory_space=pltpu.MemorySpace.VMEM),
        ],
        out_specs=pl.BlockSpec(memory_space=pltpu.MemorySpace.VMEM),
        out_shape=jax.ShapeDtypeStruct((_Q, _N), jnp.float32),
        scratch_shapes=[
            pltpu.VMEM((_CB, _N), jnp.float32),
            pltpu.VMEM((_CB, _N), jnp.float32),
            pltpu.SemaphoreType.DMA((2,)),
        ],
        compiler_params=pltpu.CompilerParams(
            vmem_limit_bytes=66998272),
    )(tft, dl_row, counts_blocks, avg)


_GATHER_DNUMS = lax.GatherDimensionNumbers(
    offset_dims=(), collapsed_slice_dims=(0,), start_index_map=(0,))


def _lane_permute(x, idx):
    return lax.gather(x, idx[:, None], _GATHER_DNUMS, slice_sizes=(1,),
                      mode=lax.GatherScatterMode.PROMISE_IN_BOUNDS)

_NG = 3125
_GCH = 196


def _topk_body2(scores_ref, vals_ref, idx_ref, buf, gbuf, cval, cidx, vv, vi):
    c = lax.axis_index("c")
    s = lax.axis_index("s")
    q = c * 16 + s

    pltpu.sync_copy(scores_ref.at[q], buf)

    neg = jnp.float32(-jnp.inf)
    lanes = lax.iota(jnp.int32, 16)
    big = jnp.int32(2**31 - 1)

    def gbody(cc, carry):
        g0 = cc * 16
        m = jnp.full((16,), neg, jnp.float32)
        for j in range(16):
            m = jnp.maximum(m, buf[pl.ds(g0 + j * _NG, 16)])
        gbuf[pl.ds(g0, 16)] = m
        return carry

    lax.fori_loop(0, _GCH - 1, gbody, 0, unroll=4)
    g0 = (_GCH - 1) * 16
    m = jnp.full((16,), neg, jnp.float32)
    for j in range(15):
        m = jnp.maximum(m, buf[pl.ds(g0 + j * _NG, 16)])
    v15 = buf[pl.ds(_N - 16, 16)]
    shift = g0 + 15 * _NG - (_N - 16)
    m15 = _lane_permute(v15, jnp.minimum(lanes + shift, 15))
    m = jnp.maximum(m, m15)
    gbuf[pl.ds(g0, 16)] = jnp.where(g0 + lanes < _NG, m, neg)

    for kk in range(_TOPK):
        def body(i, carry):
            mm, mi = carry
            v = gbuf[pl.ds(i * 16, 16)]
            upd = v > mm
            mm = jnp.where(upd, v, mm)
            mi = jnp.where(upd, i, mi)
            return mm, mi

        mm, mi = lax.fori_loop(
            0, _GCH, body,
            (jnp.full((16,), neg, jnp.float32), jnp.zeros((16,), jnp.int32)),
            unroll=8)
        mx = mm
        for sh in (8, 4, 2, 1):
            mx = jnp.maximum(mx, _lane_permute(mx, lanes ^ sh))
        cand = jnp.where(mm == mx, mi * 16 + lanes, big)
        gsel = cand
        for sh in (8, 4, 2, 1):
            gsel = jnp.minimum(gsel, _lane_permute(gsel, lanes ^ sh))
        plsc.store_scatter(gbuf, [gsel], jnp.full((16,), neg, jnp.float32),
                           mask=lanes == 0)
        didx = gsel + _NG * lanes
        cval[pl.ds(kk * 16, 16)] = plsc.load_gather(buf, [didx])
        cidx[pl.ds(kk * 16, 16)] = didx

    outv = jnp.zeros((16,), jnp.float32)
    outi = jnp.zeros((16,), jnp.int32)
    for kk in range(_TOPK):
        mm = jnp.full((16,), neg, jnp.float32)
        mi = jnp.zeros((16,), jnp.int32)
        for i in range(_TOPK):
            v = cval[pl.ds(i * 16, 16)]
            upd = v > mm
            mm = jnp.where(upd, v, mm)
            mi = jnp.where(upd, i, mi)
        mx = mm
        for sh in (8, 4, 2, 1):
            mx = jnp.maximum(mx, _lane_permute(mx, lanes ^ sh))
        cand = jnp.where(mm == mx, mi * 16 + lanes, big)
        pos = cand
        for sh in (8, 4, 2, 1):
            pos = jnp.minimum(pos, _lane_permute(pos, lanes ^ sh))
        dsel = plsc.load_gather(cidx, [pos])
        outv = jnp.where(lanes == kk, mx, outv)
        outi = jnp.where(lanes == kk, dsel, outi)
        plsc.store_scatter(cval, [pos], jnp.full((16,), neg, jnp.float32),
                           mask=lanes == 0)

    vv[...] = outv
    vi[...] = outi
    pltpu.sync_copy(vv, vals_ref.at[q])
    pltpu.sync_copy(vi, idx_ref.at[q])


def _topk_pass2(scores_t):
    mesh = plsc.VectorSubcoreMesh(core_axis_name="c", subcore_axis_name="s")
    call = functools.partial(
        pl.kernel,
        out_type=[
            jax.ShapeDtypeStruct((_Q, 16), jnp.float32),
            jax.ShapeDtypeStruct((_Q, 16), jnp.int32),
        ],
        mesh=mesh,
        scratch_types=[
            pltpu.VMEM((_N,), jnp.float32),
            pltpu.VMEM((_GCH * 16,), jnp.float32),
            pltpu.VMEM((_TOPK * 16,), jnp.float32),
            pltpu.VMEM((_TOPK * 16,), jnp.int32),
            pltpu.VMEM((16,), jnp.float32),
            pltpu.VMEM((16,), jnp.int32),
        ],
        compiler_params=pltpu.CompilerParams(needs_layout_passes=False),
    )(_topk_body2)
    return call(scores_t)


def kernel(tf, doc_len, query_terms, k):
    doc_len = doc_len.astype(jnp.float32)
    tf = tf.astype(jnp.float32)

    counts = jnp.sum(
        jax.nn.one_hot(query_terms, _V, dtype=jnp.float32), axis=1)

    avg = jnp.mean(doc_len).reshape(1, 1)
    dl_row = doc_len.reshape(1, _N)

    cbs = [counts[:, c * _CB:c * _CB + _TBH[c]] for c in range(_NCB)]
    cbs[-1] = jnp.pad(cbs[-1], ((0, 0), (0, _CB - _TBH[-1])))
    counts_blocks = jnp.stack(cbs)

    scores_t = _fused_pass(tf.T, dl_row, counts_blocks, avg)

    vals_p, idx_p = _topk_pass2(scores_t)
    vals = vals_p[:, :_TOPK]
    idx = idx_p[:, :_TOPK]
    vals = vals + 0.0 * (jnp.asarray(k, jnp.float32) - float(_TOPK))
    return vals, idx


# speedup vs baseline: 1.0697x; 1.0697x over previous
import functools

import jax
import jax.numpy as jnp
from jax import lax
from jax.experimental import pallas as pl
from jax.experimental.pallas import tpu as pltpu
from jax.experimental.pallas import tpu_sc as plsc

_K1 = 1.5
_B = 0.75
_N = 50000
_V = 1000
_Q = 32
_L = 16
_TOPK = 10


_CB = 128
_NCB = -(-_V // _CB)
_TBH = [_CB] * (_NCB - 1) + [_V - _CB * (_NCB - 1)]
_RCH = 24576
_NRCH = _N // _RCH
_TAILR = _N - _NRCH * _RCH


def _fused_body(tft_hbm, dl_ref, cnt_ref, avg_ref, out_ref, bufa, bufb, sems):
    bufs = [bufa, bufb]

    def bcopy(c, b):
        h = _TBH[c]
        src = tft_hbm.at[pl.ds(c * _CB, h), :]
        dst = bufs[b] if h == _CB else bufs[b].at[pl.ds(0, h), :]
        return pltpu.make_async_copy(src, dst, sems.at[b])

    bcopy(0, 0).start()
    avg = avg_ref[0, 0]

    for c in range(_NCB):
        h = _TBH[c]
        cur = bufs[c % 2]
        bcopy(c, c % 2).wait()
        if c + 1 < _NCB:
            bcopy(c + 1, (c + 1) % 2).start()

        def df_chunk(i, d):
            blk = cur[0:h, pl.ds(i * _RCH, _RCH)]
            return d + jnp.sum((blk > 0).astype(jnp.float32), axis=1,
                               keepdims=True)

        df = lax.fori_loop(0, _NRCH, df_chunk,
                           jnp.zeros((h, 1), jnp.float32))
        blk = cur[0:h, pl.ds(_NRCH * _RCH, _TAILR)]
        df += jnp.sum((blk > 0).astype(jnp.float32), axis=1, keepdims=True)

        idf = jnp.log((_N - df + 0.5) / (df + 0.5))
        cnt = cnt_ref[c][:, 0:h]

        def sc_chunk(r0, rn):
            tfb = cur[0:h, pl.ds(r0, rn)]
            dlr = dl_ref[0:1, pl.ds(r0, rn)]
            norm = _K1 * (1.0 - _B + _B * dlr / avg)
            num = tfb * (_K1 + 1.0)
            ap = idf * num / (tfb + norm)
            return lax.dot_general(
                cnt, ap, (((1,), (0,)), ((), ())),
                preferred_element_type=jnp.float32,
                precision=lax.Precision.HIGHEST)

        if c == 0:
            def body0(i, _):
                r0 = i * _RCH
                out_ref[:, pl.ds(r0, _RCH)] = sc_chunk(r0, _RCH)
                return 0
            lax.fori_loop(0, _NRCH, body0, 0)
            out_ref[:, pl.ds(_NRCH * _RCH, _TAILR)] = sc_chunk(
                _NRCH * _RCH, _TAILR)
        else:
            def bodyn(i, _):
                r0 = i * _RCH
                out_ref[:, pl.ds(r0, _RCH)] += sc_chunk(r0, _RCH)
                return 0
            lax.fori_loop(0, _NRCH, bodyn, 0)
            out_ref[:, pl.ds(_NRCH * _RCH, _TAILR)] += sc_chunk(
                _NRCH * _RCH, _TAILR)


def _fused_pass(tft, dl_row, counts_blocks, avg):
    return pl.pallas_call(
        _fused_body,
        in_specs=[
            pl.BlockSpec(memory_space=pl.ANY),
            pl.BlockSpec(memory_space=pltpu.MemorySpace.VMEM),
            pl.BlockSpec(memory_space=pltpu.MemorySpace.VMEM),
            pl.BlockSpec(memory_space=pltpu.MemorySpace.VMEM),
        ],
        out_specs=pl.BlockSpec(memory_space=pltpu.MemorySpace.VMEM),
        out_shape=jax.ShapeDtypeStruct((_Q, _N), jnp.float32),
        scratch_shapes=[
            pltpu.VMEM((_CB, _N), jnp.float32),
            pltpu.VMEM((_CB, _N), jnp.float32),
            pltpu.SemaphoreType.DMA((2,)),
        ],
        compiler_params=pltpu.CompilerParams(
            vmem_limit_bytes=63 * 1024 * 1024),
    )(tft, dl_row, counts_blocks, avg)


_GATHER_DNUMS = lax.GatherDimensionNumbers(
    offset_dims=(), collapsed_slice_dims=(0,), start_index_map=(0,))


def _lane_permute(x, idx):
    return lax.gather(x, idx[:, None], _GATHER_DNUMS, slice_sizes=(1,),
                      mode=lax.GatherScatterMode.PROMISE_IN_BOUNDS)

_NG = 3125
_GCH = 196


def _topk_body2(scores_ref, vals_ref, idx_ref, buf, gbuf, cval, cidx, vv, vi):
    c = lax.axis_index("c")
    s = lax.axis_index("s")
    q = c * 16 + s

    pltpu.sync_copy(scores_ref.at[q], buf)

    neg = jnp.float32(-jnp.inf)
    lanes = lax.iota(jnp.int32, 16)
    big = jnp.int32(2**31 - 1)

    def gbody(cc, carry):
        g0 = cc * 16
        m = jnp.full((16,), neg, jnp.float32)
        for j in range(16):
            m = jnp.maximum(m, buf[pl.ds(g0 + j * _NG, 16)])
        gbuf[pl.ds(g0, 16)] = m
        return carry

    lax.fori_loop(0, _GCH - 1, gbody, 0, unroll=4)
    g0 = (_GCH - 1) * 16
    m = jnp.full((16,), neg, jnp.float32)
    for j in range(15):
        m = jnp.maximum(m, buf[pl.ds(g0 + j * _NG, 16)])
    v15 = buf[pl.ds(_N - 16, 16)]
    shift = g0 + 15 * _NG - (_N - 16)
    m15 = _lane_permute(v15, jnp.minimum(lanes + shift, 15))
    m = jnp.maximum(m, m15)
    gbuf[pl.ds(g0, 16)] = jnp.where(g0 + lanes < _NG, m, neg)

    for kk in range(_TOPK):
        def body(i, carry):
            mm, mi = carry
            v = gbuf[pl.ds(i * 16, 16)]
            upd = v > mm
            mm = jnp.where(upd, v, mm)
            mi = jnp.where(upd, i, mi)
            return mm, mi

        mm, mi = lax.fori_loop(
            0, _GCH, body,
            (jnp.full((16,), neg, jnp.float32), jnp.zeros((16,), jnp.int32)),
            unroll=8)
        mx = mm
        for sh in (8, 4, 2, 1):
            mx = jnp.maximum(mx, _lane_permute(mx, lanes ^ sh))
        cand = jnp.where(mm == mx, mi * 16 + lanes, big)
        gsel = cand
        for sh in (8, 4, 2, 1):
            gsel = jnp.minimum(gsel, _lane_permute(gsel, lanes ^ sh))
        plsc.store_scatter(gbuf, [gsel], jnp.full((16,), neg, jnp.float32),
                           mask=lanes == 0)
        didx = gsel + _NG * lanes
        cval[pl.ds(kk * 16, 16)] = plsc.load_gather(buf, [didx])
        cidx[pl.ds(kk * 16, 16)] = didx

    outv = jnp.zeros((16,), jnp.float32)
    outi = jnp.zeros((16,), jnp.int32)
    for kk in range(_TOPK):
        mm = jnp.full((16,), neg, jnp.float32)
        mi = jnp.zeros((16,), jnp.int32)
        for i in range(_TOPK):
            v = cval[pl.ds(i * 16, 16)]
            upd = v > mm
            mm = jnp.where(upd, v, mm)
            mi = jnp.where(upd, i, mi)
        mx = mm
        for sh in (8, 4, 2, 1):
            mx = jnp.maximum(mx, _lane_permute(mx, lanes ^ sh))
        cand = jnp.where(mm == mx, mi * 16 + lanes, big)
        pos = cand
        for sh in (8, 4, 2, 1):
            pos = jnp.minimum(pos, _lane_permute(pos, lanes ^ sh))
        dsel = plsc.load_gather(cidx, [pos])
        outv = jnp.where(lanes == kk, mx, outv)
        outi = jnp.where(lanes == kk, dsel, outi)
        plsc.store_scatter(cval, [pos], jnp.full((16,), neg, jnp.float32),
                           mask=lanes == 0)

    vv[...] = outv
    vi[...] = outi
    pltpu.sync_copy(vv, vals_ref.at[q])
    pltpu.sync_copy(vi, idx_ref.at[q])


def _topk_pass2(scores_t):
    mesh = plsc.VectorSubcoreMesh(core_axis_name="c", subcore_axis_name="s")
    call = functools.partial(
        pl.kernel,
        out_type=[
            jax.ShapeDtypeStruct((_Q, 16), jnp.float32),
            jax.ShapeDtypeStruct((_Q, 16), jnp.int32),
        ],
        mesh=mesh,
        scratch_types=[
            pltpu.VMEM((_N,), jnp.float32),
            pltpu.VMEM((_GCH * 16,), jnp.float32),
            pltpu.VMEM((_TOPK * 16,), jnp.float32),
            pltpu.VMEM((_TOPK * 16,), jnp.int32),
            pltpu.VMEM((16,), jnp.float32),
            pltpu.VMEM((16,), jnp.int32),
        ],
        compiler_params=pltpu.CompilerParams(needs_layout_passes=False),
    )(_topk_body2)
    return call(scores_t)


def kernel(tf, doc_len, query_terms, k):
    doc_len = doc_len.astype(jnp.float32)
    tf = tf.astype(jnp.float32)

    counts = jnp.sum(
        jax.nn.one_hot(query_terms, _V, dtype=jnp.float32), axis=1)

    avg = jnp.mean(doc_len).reshape(1, 1)
    dl_row = doc_len.reshape(1, _N)

    cbs = [counts[:, c * _CB:c * _CB + _TBH[c]] for c in range(_NCB)]
    cbs[-1] = jnp.pad(cbs[-1], ((0, 0), (0, _CB - _TBH[-1])))
    counts_blocks = jnp.stack(cbs)

    scores_t = _fused_pass(tf.T, dl_row, counts_blocks, avg)

    vals_p, idx_p = _topk_pass2(scores_t)
    vals = vals_p[:, :_TOPK]
    idx = idx_p[:, :_TOPK]
    vals = vals + 0.0 * (jnp.asarray(k, jnp.float32) - float(_TOPK))
    return vals, idx
